# baseline (speedup 1.0000x reference)
import functools

import jax
import jax.numpy as jnp
from jax import lax
from jax.experimental import pallas as pl
from jax.experimental.pallas import tpu as pltpu
from jax.experimental.pallas import tpu_sc as plsc

NC = 2
NS = 16
NW = NC * NS
LANES = 16

B = 2
NX = 128
PLANE = 128 * 128
NVOX = NX * PLANE
CH = 7
X_SC = 64
X_TC = NX - X_SC
QS = 4
VC = PLANE // QS
CPW = X_SC * QS // NW
TOT = B * CPW
GROUPS = VC // LANES

_MAX_CH = (0, 1, 6)
_MEAN_CH = (2, 3, 4, 5)
P_ROWS = 3 * B * CH
P_FLAT = P_ROWS * LANES


def _sc_body(pred_hbm, mask_hbm, out_hbm, mb0, mb1, pb0, pb1, obuf,
             sm0, sm1, sp0, sp1):
  cid = lax.axis_index("c")
  sid = lax.axis_index("s")
  w = sid * NC + cid
  c0 = w * CPW

  mbufs = (mb0, mb1)
  pbufs = (pb0, pb1)
  msems = (sm0, sm1)
  psems = (sp0, sp1)

  neg_inf = jnp.float32(-jnp.inf)
  zero = jnp.zeros((LANES,), jnp.float32)
  one = jnp.float32(1.0)

  def start(step):
    b, rem = divmod(step, CPW)
    par = step % 2
    c = c0 + rem
    xi = c // QS
    q = c % QS
    prow = b * NX + xi
    poff = prow * PLANE + q * VC
    moff = prow * CH * PLANE + q * VC
    copies = [
        pltpu.async_copy(mask_hbm.at[pl.ds(moff + ch * PLANE, VC)],
                         mbufs[par].at[ch], msems[par])
        for ch in range(CH)
    ]
    copies.append(pltpu.async_copy(pred_hbm.at[pl.ds(poff, VC)],
                                   pbufs[par], psems[par]))
    return copies

  pending = start(0)
  cnts = sums = maxs = None
  for step in range(TOT):
    b, k = divmod(step, CPW)
    par = step % 2
    if k == 0:
      cnts = [zero] * CH
      sums = {ch: zero for ch in _MEAN_CH}
      maxs = {ch: jnp.full((LANES,), neg_inf) for ch in _MAX_CH}
    nxt = start(step + 1) if step + 1 < TOT else None
    for c in pending:
      c.wait()
    pending = nxt
    mb = mbufs[par]
    pb = pbufs[par]

    def inner(g, carry, mb=mb, pb=pb):
      cnts = list(carry[:CH])
      sums = dict(zip(_MEAN_CH, carry[CH:CH + 4]))
      maxs = dict(zip(_MAX_CH, carry[CH + 4:]))
      off = g * LANES
      pv = pb[pl.ds(off, LANES)]
      for ch in range(CH):
        mv = mb[ch, pl.ds(off, LANES)]
        m = mv > 0.0
        cnts[ch] = cnts[ch] + jnp.where(m, one, 0.0)
        if ch in _MEAN_CH:
          sums[ch] = sums[ch] + jnp.where(m, pv, 0.0)
        else:
          maxs[ch] = jnp.maximum(maxs[ch], jnp.where(m, pv, neg_inf))
      return tuple(cnts) + tuple(sums[c] for c in _MEAN_CH) + tuple(
          maxs[c] for c in _MAX_CH)

    carry = tuple(cnts) + tuple(sums[c] for c in _MEAN_CH) + tuple(
        maxs[c] for c in _MAX_CH)
    carry = lax.fori_loop(0, GROUPS, inner, carry)
    cnts = list(carry[:CH])
    sums = dict(zip(_MEAN_CH, carry[CH:CH + 4]))
    maxs = dict(zip(_MAX_CH, carry[CH + 4:]))

    if k == CPW - 1:
      for ch in range(CH):
        r0 = (0 * B * CH + b * CH + ch) * LANES
        r1 = (1 * B * CH + b * CH + ch) * LANES
        r2 = (2 * B * CH + b * CH + ch) * LANES
        obuf[pl.ds(r0, LANES)] = sums[ch] if ch in _MEAN_CH else zero
        obuf[pl.ds(r1, LANES)] = cnts[ch]
        obuf[pl.ds(r2, LANES)] = (
            maxs[ch] if ch in _MAX_CH else jnp.full((LANES,), neg_inf))

  pltpu.sync_copy(obuf, out_hbm.at[pl.ds(w * P_FLAT, P_FLAT)])


_sc_partials = functools.partial(
    pl.kernel,
    out_type=jax.ShapeDtypeStruct((NW * P_FLAT,), jnp.float32),
    mesh=plsc.VectorSubcoreMesh(core_axis_name="c", subcore_axis_name="s",
                                num_cores=NC, num_subcores=NS),
    scratch_types=[
        pltpu.VMEM((CH, VC), jnp.float32),
        pltpu.VMEM((CH, VC), jnp.float32),
        pltpu.VMEM((VC,), jnp.float32),
        pltpu.VMEM((VC,), jnp.float32),
        pltpu.VMEM((P_FLAT,), jnp.float32),
        pltpu.SemaphoreType.DMA,
        pltpu.SemaphoreType.DMA,
        pltpu.SemaphoreType.DMA,
        pltpu.SemaphoreType.DMA,
    ],
    compiler_params=pltpu.CompilerParams(use_tc_tiling_on_sc=False,
                                         needs_layout_passes=False),
)(_sc_body)

_THRESH = {0: 54.0, 1: 48.0, 2: 26.0, 3: 26.0, 4: 45.0, 5: 45.0, 6: 73.5}


def _tc_body(pred_ref, mask_ref, out_ref):
  pv = pred_ref[0]
  neg_inf = jnp.float32(-jnp.inf)
  vals = {}
  for ch in range(CH):
    mv = mask_ref[ch]
    m = mv > 0.0
    vals[ch] = jnp.sum(jnp.where(m, 1.0, 0.0))
    if ch in _MEAN_CH:
      vals[CH + ch] = jnp.sum(jnp.where(m, pv, 0.0))
    else:
      vals[2 * CH + ch] = jnp.max(jnp.where(m, pv, neg_inf))
  col = lax.broadcasted_iota(jnp.int32, (1, 1, 3 * CH), 2)
  row = jnp.zeros((1, 1, 3 * CH), jnp.float32)
  for k, v in vals.items():
    row = jnp.where(col == k, v, row)
  out_ref[...] = row


_tc_call = pl.pallas_call(
    _tc_body,
    grid=(B, X_TC),
    in_specs=[
        pl.BlockSpec((1, 128, 128), lambda b, x: (b * NX + X_SC + x, 0, 0)),
        pl.BlockSpec((CH, 128, 128), lambda b, x: (b * NX + X_SC + x, 0, 0)),
    ],
    out_specs=pl.BlockSpec((1, 1, 3 * CH), lambda b, x: (b * X_TC + x, 0, 0)),
    out_shape=jax.ShapeDtypeStruct((B * X_TC, 1, 3 * CH), jnp.float32),
)


def _combine(p_ref, t_ref, o_ref):
  p = p_ref[...]
  t = t_ref[...][:, 0, :]

  def seg_sum(stat, b, ch):
    off = (stat * B * CH + b * CH + ch) * LANES
    return jnp.sum(p[:, off:off + LANES])

  def seg_max(stat, b, ch):
    off = (stat * B * CH + b * CH + ch) * LANES
    return jnp.max(p[:, off:off + LANES])

  def tc_sum(col, b):
    return jnp.sum(t[b * X_TC:(b + 1) * X_TC, col:col + 1])

  def tc_max(col, b):
    return jnp.max(t[b * X_TC:(b + 1) * X_TC, col:col + 1])

  total = jnp.float32(0.0)
  count = jnp.float32(0.0)
  for ch in range(CH):
    struct_loss = jnp.float32(0.0)
    present_any = jnp.float32(0.0)
    for b in range(B):
      cnt = seg_sum(1, b, ch) + tc_sum(ch, b)
      present = cnt > 0.0
      if ch in _MAX_CH:
        stat = jnp.maximum(seg_max(2, b, ch), tc_max(2 * CH + ch, b))
      else:
        stat = (seg_sum(0, b, ch) + tc_sum(CH + ch, b)) / jnp.maximum(cnt, 1.0)
      loss_b = jnp.where(stat <= jnp.float32(_THRESH[ch]), 0.0, 1.0)
      struct_loss = struct_loss + jnp.where(present, loss_b, 0.0)
      present_any = jnp.maximum(present_any,
                                jnp.where(present, 1.0, 0.0))
    total = total + struct_loss
    count = count + present_any
  o_ref[0] = total / jnp.maximum(count, 1.0)


_combine_call = pl.pallas_call(
    _combine,
    out_shape=jax.ShapeDtypeStruct((1,), jnp.float32),
    out_specs=pl.BlockSpec(memory_space=pltpu.SMEM),
)


def kernel(predicted, structure_masks):
  pred_flat = predicted.reshape(-1)
  mask_nat = structure_masks.transpose(0, 1, 4, 2, 3)
  mask_lin = mask_nat.reshape(-1)
  pred3d = predicted.reshape(B * NX, 128, 128)
  mask3d = mask_nat.reshape(B * NX * CH, 128, 128)
  partials = _sc_partials(pred_flat, mask_lin)
  tparts = _tc_call(pred3d, mask3d)
  partials = partials.reshape(NW, P_FLAT)
  return _combine_call(partials, tparts).astype(predicted.dtype)


# speedup vs baseline: 1.4280x; 1.4280x over previous
import functools

import jax
import jax.numpy as jnp
from jax import lax
from jax.experimental import pallas as pl
from jax.experimental.pallas import tpu as pltpu
from jax.experimental.pallas import tpu_sc as plsc

NC = 2
NS = 16
NW = NC * NS
LANES = 16

B = 2
NX = 128
PLANE = 128 * 128
NVOX = NX * PLANE
CH = 7
X_SC = 96
X_TC = NX - X_SC
TC_XS = 2
QS = 4
VC = PLANE // QS
CPW = X_SC * QS // NW
TOT = B * CPW
GROUPS = VC // LANES

_MAX_CH = (0, 1, 6)
_MEAN_CH = (2, 3, 4, 5)
P_ROWS = 3 * B * CH
P_FLAT = P_ROWS * LANES


def _sc_body(pred_hbm, mask_hbm, out_hbm, mb0, mb1, pb0, pb1, obuf,
             sm0, sm1, sp0, sp1):
  cid = lax.axis_index("c")
  sid = lax.axis_index("s")
  w = sid * NC + cid
  c0 = w * CPW

  mbufs = (mb0, mb1)
  pbufs = (pb0, pb1)
  msems = (sm0, sm1)
  psems = (sp0, sp1)

  neg_inf = jnp.float32(-jnp.inf)
  zero = jnp.zeros((LANES,), jnp.float32)
  one = jnp.float32(1.0)

  def start(step):
    b, rem = divmod(step, CPW)
    par = step % 2
    c = c0 + rem
    xi = c // QS
    q = c % QS
    prow = b * NX + xi
    poff = prow * PLANE + q * VC
    moff = prow * CH * PLANE + q * VC
    copies = [
        pltpu.async_copy(mask_hbm.at[pl.ds(moff + ch * PLANE, VC)],
                         mbufs[par].at[ch], msems[par])
        for ch in range(CH)
    ]
    copies.append(pltpu.async_copy(pred_hbm.at[pl.ds(poff, VC)],
                                   pbufs[par], psems[par]))
    return copies

  pending = start(0)
  cnts = sums = maxs = None
  for step in range(TOT):
    b, k = divmod(step, CPW)
    par = step % 2
    if k == 0:
      cnts = [zero] * CH
      sums = {ch: zero for ch in _MEAN_CH}
      maxs = {ch: jnp.full((LANES,), neg_inf) for ch in _MAX_CH}
    nxt = start(step + 1) if step + 1 < TOT else None
    for c in pending:
      c.wait()
    pending = nxt
    mb = mbufs[par]
    pb = pbufs[par]

    def inner(g, carry, mb=mb, pb=pb):
      cnts = list(carry[:CH])
      sums = dict(zip(_MEAN_CH, carry[CH:CH + 4]))
      maxs = dict(zip(_MAX_CH, carry[CH + 4:]))
      off = g * LANES
      pv = pb[pl.ds(off, LANES)]
      for ch in range(CH):
        mv = mb[ch, pl.ds(off, LANES)]
        m = mv > 0.0
        cnts[ch] = cnts[ch] + jnp.where(m, one, 0.0)
        if ch in _MEAN_CH:
          sums[ch] = sums[ch] + jnp.where(m, pv, 0.0)
        else:
          maxs[ch] = jnp.maximum(maxs[ch], jnp.where(m, pv, neg_inf))
      return tuple(cnts) + tuple(sums[c] for c in _MEAN_CH) + tuple(
          maxs[c] for c in _MAX_CH)

    carry = tuple(cnts) + tuple(sums[c] for c in _MEAN_CH) + tuple(
        maxs[c] for c in _MAX_CH)
    carry = lax.fori_loop(0, GROUPS, inner, carry)
    cnts = list(carry[:CH])
    sums = dict(zip(_MEAN_CH, carry[CH:CH + 4]))
    maxs = dict(zip(_MAX_CH, carry[CH + 4:]))

    if k == CPW - 1:
      for ch in range(CH):
        r0 = (0 * B * CH + b * CH + ch) * LANES
        r1 = (1 * B * CH + b * CH + ch) * LANES
        r2 = (2 * B * CH + b * CH + ch) * LANES
        obuf[pl.ds(r0, LANES)] = sums[ch] if ch in _MEAN_CH else zero
        obuf[pl.ds(r1, LANES)] = cnts[ch]
        obuf[pl.ds(r2, LANES)] = (
            maxs[ch] if ch in _MAX_CH else jnp.full((LANES,), neg_inf))

  pltpu.sync_copy(obuf, out_hbm.at[pl.ds(w * P_FLAT, P_FLAT)])


_sc_partials = functools.partial(
    pl.kernel,
    out_type=jax.ShapeDtypeStruct((NW * P_FLAT,), jnp.float32),
    mesh=plsc.VectorSubcoreMesh(core_axis_name="c", subcore_axis_name="s",
                                num_cores=NC, num_subcores=NS),
    scratch_types=[
        pltpu.VMEM((CH, VC), jnp.float32),
        pltpu.VMEM((CH, VC), jnp.float32),
        pltpu.VMEM((VC,), jnp.float32),
        pltpu.VMEM((VC,), jnp.float32),
        pltpu.VMEM((P_FLAT,), jnp.float32),
        pltpu.SemaphoreType.DMA,
        pltpu.SemaphoreType.DMA,
        pltpu.SemaphoreType.DMA,
        pltpu.SemaphoreType.DMA,
    ],
    compiler_params=pltpu.CompilerParams(use_tc_tiling_on_sc=False,
                                         needs_layout_passes=False),
)(_sc_body)

_THRESH = {0: 54.0, 1: 48.0, 2: 26.0, 3: 26.0, 4: 45.0, 5: 45.0, 6: 73.5}


def _tc_body(pred_ref, mask_ref, out_ref):
  neg_inf = jnp.float32(-jnp.inf)
  vals = {}
  for xs in range(TC_XS):
    pv = pred_ref[xs]
    for ch in range(CH):
      mv = mask_ref[xs * CH + ch]
      m = mv > 0.0
      cnt = jnp.sum(jnp.where(m, 1.0, 0.0))
      vals[ch] = vals.get(ch, 0.0) + cnt
      if ch in _MEAN_CH:
        s = jnp.sum(jnp.where(m, pv, 0.0))
        vals[CH + ch] = vals.get(CH + ch, 0.0) + s
      else:
        mx = jnp.max(jnp.where(m, pv, neg_inf))
        vals[2 * CH + ch] = jnp.maximum(vals.get(2 * CH + ch, neg_inf), mx)
  col = lax.broadcasted_iota(jnp.int32, (1, 1, 3 * CH), 2)
  row = jnp.zeros((1, 1, 3 * CH), jnp.float32)
  for k, v in vals.items():
    row = jnp.where(col == k, v, row)
  out_ref[...] = row


_TC_STEPS = X_TC // TC_XS

_tc_call = pl.pallas_call(
    _tc_body,
    grid=(B, _TC_STEPS),
    in_specs=[
        pl.BlockSpec((TC_XS, 128, 128),
                     lambda b, x: ((b * NX + X_SC) // TC_XS + x, 0, 0)),
        pl.BlockSpec((TC_XS * CH, 128, 128),
                     lambda b, x: ((b * NX + X_SC) // TC_XS + x, 0, 0)),
    ],
    out_specs=pl.BlockSpec((1, 1, 3 * CH),
                           lambda b, x: (b * _TC_STEPS + x, 0, 0)),
    out_shape=jax.ShapeDtypeStruct((B * _TC_STEPS, 1, 3 * CH), jnp.float32),
)


def _combine(p_ref, t_ref, o_ref):
  p = p_ref[...]
  t = t_ref[...][:, 0, :]

  def seg_sum(stat, b, ch):
    off = (stat * B * CH + b * CH + ch) * LANES
    return jnp.sum(p[:, off:off + LANES])

  def seg_max(stat, b, ch):
    off = (stat * B * CH + b * CH + ch) * LANES
    return jnp.max(p[:, off:off + LANES])

  def tc_sum(col, b):
    return jnp.sum(t[b * _TC_STEPS:(b + 1) * _TC_STEPS, col:col + 1])

  def tc_max(col, b):
    return jnp.max(t[b * _TC_STEPS:(b + 1) * _TC_STEPS, col:col + 1])

  total = jnp.float32(0.0)
  count = jnp.float32(0.0)
  for ch in range(CH):
    struct_loss = jnp.float32(0.0)
    present_any = jnp.float32(0.0)
    for b in range(B):
      cnt = seg_sum(1, b, ch) + tc_sum(ch, b)
      present = cnt > 0.0
      if ch in _MAX_CH:
        stat = jnp.maximum(seg_max(2, b, ch), tc_max(2 * CH + ch, b))
      else:
        stat = (seg_sum(0, b, ch) + tc_sum(CH + ch, b)) / jnp.maximum(cnt, 1.0)
      loss_b = jnp.where(stat <= jnp.float32(_THRESH[ch]), 0.0, 1.0)
      struct_loss = struct_loss + jnp.where(present, loss_b, 0.0)
      present_any = jnp.maximum(present_any,
                                jnp.where(present, 1.0, 0.0))
    total = total + struct_loss
    count = count + present_any
  o_ref[0] = total / jnp.maximum(count, 1.0)


_combine_call = pl.pallas_call(
    _combine,
    out_shape=jax.ShapeDtypeStruct((1,), jnp.float32),
    out_specs=pl.BlockSpec(memory_space=pltpu.SMEM),
)


def kernel(predicted, structure_masks):
  pred_flat = predicted.reshape(-1)
  mask_nat = structure_masks.transpose(0, 1, 4, 2, 3)
  mask_lin = mask_nat.reshape(-1)
  pred3d = predicted.reshape(B * NX, 128, 128)
  mask3d = mask_nat.reshape(B * NX * CH, 128, 128)
  partials = _sc_partials(pred_flat, mask_lin)
  tparts = _tc_call(pred3d, mask3d)
  partials = partials.reshape(NW, P_FLAT)
  return _combine_call(partials, tparts).astype(predicted.dtype)


# speedup vs baseline: 1.4604x; 1.0227x over previous
import functools

import jax
import jax.numpy as jnp
from jax import lax
from jax.experimental import pallas as pl
from jax.experimental.pallas import tpu as pltpu
from jax.experimental.pallas import tpu_sc as plsc

NC = 2
NS = 16
NW = NC * NS
LANES = 16

B = 2
NX = 128
PLANE = 128 * 128
NVOX = NX * PLANE
CH = 7
X_SC = 88
X_TC = NX - X_SC
TC_XS = 4
QS = 4
VC = PLANE // QS
CPW = X_SC * QS // NW
TOT = B * CPW
GROUPS = VC // LANES

_MAX_CH = (0, 1, 6)
_MEAN_CH = (2, 3, 4, 5)
P_ROWS = 3 * B * CH
P_FLAT = P_ROWS * LANES


def _sc_body(pred_hbm, mask_hbm, out_hbm, mb0, mb1, pb0, pb1, obuf,
             sm0, sm1, sp0, sp1):
  cid = lax.axis_index("c")
  sid = lax.axis_index("s")
  w = sid * NC + cid
  c0 = w * CPW

  mbufs = (mb0, mb1)
  pbufs = (pb0, pb1)
  msems = (sm0, sm1)
  psems = (sp0, sp1)

  neg_inf = jnp.float32(-jnp.inf)
  zero = jnp.zeros((LANES,), jnp.float32)
  one = jnp.float32(1.0)

  def start(step):
    b, rem = divmod(step, CPW)
    par = step % 2
    c = c0 + rem
    xi = c // QS
    q = c % QS
    prow = b * NX + xi
    poff = prow * PLANE + q * VC
    moff = prow * CH * PLANE + q * VC
    copies = [
        pltpu.async_copy(mask_hbm.at[pl.ds(moff + ch * PLANE, VC)],
                         mbufs[par].at[ch], msems[par])
        for ch in range(CH)
    ]
    copies.append(pltpu.async_copy(pred_hbm.at[pl.ds(poff, VC)],
                                   pbufs[par], psems[par]))
    return copies

  pending = start(0)
  cnts = sums = maxs = None
  for step in range(TOT):
    b, k = divmod(step, CPW)
    par = step % 2
    if k == 0:
      cnts = [zero] * CH
      sums = {ch: zero for ch in _MEAN_CH}
      maxs = {ch: jnp.full((LANES,), neg_inf) for ch in _MAX_CH}
    nxt = start(step + 1) if step + 1 < TOT else None
    for c in pending:
      c.wait()
    pending = nxt
    mb = mbufs[par]
    pb = pbufs[par]

    def inner(g, carry, mb=mb, pb=pb):
      cnts = list(carry[:CH])
      sums = dict(zip(_MEAN_CH, carry[CH:CH + 4]))
      maxs = dict(zip(_MAX_CH, carry[CH + 4:]))
      off = g * LANES
      pv = pb[pl.ds(off, LANES)]
      for ch in range(CH):
        mv = mb[ch, pl.ds(off, LANES)]
        m = mv > 0.0
        cnts[ch] = cnts[ch] + jnp.where(m, one, 0.0)
        if ch in _MEAN_CH:
          sums[ch] = sums[ch] + jnp.where(m, pv, 0.0)
        else:
          maxs[ch] = jnp.maximum(maxs[ch], jnp.where(m, pv, neg_inf))
      return tuple(cnts) + tuple(sums[c] for c in _MEAN_CH) + tuple(
          maxs[c] for c in _MAX_CH)

    carry = tuple(cnts) + tuple(sums[c] for c in _MEAN_CH) + tuple(
        maxs[c] for c in _MAX_CH)
    carry = lax.fori_loop(0, GROUPS, inner, carry)
    cnts = list(carry[:CH])
    sums = dict(zip(_MEAN_CH, carry[CH:CH + 4]))
    maxs = dict(zip(_MAX_CH, carry[CH + 4:]))

    if k == CPW - 1:
      for ch in range(CH):
        r0 = (0 * B * CH + b * CH + ch) * LANES
        r1 = (1 * B * CH + b * CH + ch) * LANES
        r2 = (2 * B * CH + b * CH + ch) * LANES
        obuf[pl.ds(r0, LANES)] = sums[ch] if ch in _MEAN_CH else zero
        obuf[pl.ds(r1, LANES)] = cnts[ch]
        obuf[pl.ds(r2, LANES)] = (
            maxs[ch] if ch in _MAX_CH else jnp.full((LANES,), neg_inf))

  pltpu.sync_copy(obuf, out_hbm.at[pl.ds(w * P_FLAT, P_FLAT)])


_sc_partials = functools.partial(
    pl.kernel,
    out_type=jax.ShapeDtypeStruct((NW * P_FLAT,), jnp.float32),
    mesh=plsc.VectorSubcoreMesh(core_axis_name="c", subcore_axis_name="s",
                                num_cores=NC, num_subcores=NS),
    scratch_types=[
        pltpu.VMEM((CH, VC), jnp.float32),
        pltpu.VMEM((CH, VC), jnp.float32),
        pltpu.VMEM((VC,), jnp.float32),
        pltpu.VMEM((VC,), jnp.float32),
        pltpu.VMEM((P_FLAT,), jnp.float32),
        pltpu.SemaphoreType.DMA,
        pltpu.SemaphoreType.DMA,
        pltpu.SemaphoreType.DMA,
        pltpu.SemaphoreType.DMA,
    ],
    compiler_params=pltpu.CompilerParams(use_tc_tiling_on_sc=False,
                                         needs_layout_passes=False),
)(_sc_body)

_THRESH = {0: 54.0, 1: 48.0, 2: 26.0, 3: 26.0, 4: 45.0, 5: 45.0, 6: 73.5}


def _tc_body(pred_ref, mask_ref, out_ref):
  neg_inf = jnp.float32(-jnp.inf)
  vals = {}
  for xs in range(TC_XS):
    pv = pred_ref[xs]
    for ch in range(CH):
      mv = mask_ref[xs * CH + ch]
      m = mv > 0.0
      cnt = jnp.sum(jnp.where(m, 1.0, 0.0))
      vals[ch] = vals.get(ch, 0.0) + cnt
      if ch in _MEAN_CH:
        s = jnp.sum(jnp.where(m, pv, 0.0))
        vals[CH + ch] = vals.get(CH + ch, 0.0) + s
      else:
        mx = jnp.max(jnp.where(m, pv, neg_inf))
        vals[2 * CH + ch] = jnp.maximum(vals.get(2 * CH + ch, neg_inf), mx)
  col = lax.broadcasted_iota(jnp.int32, (1, 1, 3 * CH), 2)
  row = jnp.zeros((1, 1, 3 * CH), jnp.float32)
  for k, v in vals.items():
    row = jnp.where(col == k, v, row)
  out_ref[...] = row


_TC_STEPS = X_TC // TC_XS

_tc_call = pl.pallas_call(
    _tc_body,
    grid=(B, _TC_STEPS),
    in_specs=[
        pl.BlockSpec((TC_XS, 128, 128),
                     lambda b, x: ((b * NX + X_SC) // TC_XS + x, 0, 0)),
        pl.BlockSpec((TC_XS * CH, 128, 128),
                     lambda b, x: ((b * NX + X_SC) // TC_XS + x, 0, 0)),
    ],
    out_specs=pl.BlockSpec((1, 1, 3 * CH),
                           lambda b, x: (b * _TC_STEPS + x, 0, 0)),
    out_shape=jax.ShapeDtypeStruct((B * _TC_STEPS, 1, 3 * CH), jnp.float32),
)


def _combine(p_ref, t_ref, o_ref):
  p = p_ref[...]
  t = t_ref[...][:, 0, :]

  def seg_sum(stat, b, ch):
    off = (stat * B * CH + b * CH + ch) * LANES
    return jnp.sum(p[:, off:off + LANES])

  def seg_max(stat, b, ch):
    off = (stat * B * CH + b * CH + ch) * LANES
    return jnp.max(p[:, off:off + LANES])

  def tc_sum(col, b):
    return jnp.sum(t[b * _TC_STEPS:(b + 1) * _TC_STEPS, col:col + 1])

  def tc_max(col, b):
    return jnp.max(t[b * _TC_STEPS:(b + 1) * _TC_STEPS, col:col + 1])

  total = jnp.float32(0.0)
  count = jnp.float32(0.0)
  for ch in range(CH):
    struct_loss = jnp.float32(0.0)
    present_any = jnp.float32(0.0)
    for b in range(B):
      cnt = seg_sum(1, b, ch) + tc_sum(ch, b)
      present = cnt > 0.0
      if ch in _MAX_CH:
        stat = jnp.maximum(seg_max(2, b, ch), tc_max(2 * CH + ch, b))
      else:
        stat = (seg_sum(0, b, ch) + tc_sum(CH + ch, b)) / jnp.maximum(cnt, 1.0)
      loss_b = jnp.where(stat <= jnp.float32(_THRESH[ch]), 0.0, 1.0)
      struct_loss = struct_loss + jnp.where(present, loss_b, 0.0)
      present_any = jnp.maximum(present_any,
                                jnp.where(present, 1.0, 0.0))
    total = total + struct_loss
    count = count + present_any
  o_ref[0] = total / jnp.maximum(count, 1.0)


_combine_call = pl.pallas_call(
    _combine,
    out_shape=jax.ShapeDtypeStruct((1,), jnp.float32),
    out_specs=pl.BlockSpec(memory_space=pltpu.SMEM),
)


def kernel(predicted, structure_masks):
  pred_flat = predicted.reshape(-1)
  mask_nat = structure_masks.transpose(0, 1, 4, 2, 3)
  mask_lin = mask_nat.reshape(-1)
  pred3d = predicted.reshape(B * NX, 128, 128)
  mask3d = mask_nat.reshape(B * NX * CH, 128, 128)
  partials = _sc_partials(pred_flat, mask_lin)
  tparts = _tc_call(pred3d, mask3d)
  partials = partials.reshape(NW, P_FLAT)
  return _combine_call(partials, tparts).astype(predicted.dtype)


# speedup vs baseline: 1.5655x; 1.0720x over previous
import functools

import jax
import jax.numpy as jnp
from jax import lax
from jax.experimental import pallas as pl
from jax.experimental.pallas import tpu as pltpu
from jax.experimental.pallas import tpu_sc as plsc

NC = 2
NS = 16
NW = NC * NS
LANES = 16

B = 2
NX = 128
PLANE = 128 * 128
NVOX = NX * PLANE
CH = 7
X_SC = 72
X_TC = NX - X_SC
TC_XS = 8
QS = 4
VC = PLANE // QS
CPW = X_SC * QS // NW
TOT = B * CPW
GROUPS = VC // LANES

_MAX_CH = (0, 1, 6)
_MEAN_CH = (2, 3, 4, 5)
P_ROWS = 3 * B * CH
P_FLAT = P_ROWS * LANES


def _sc_body(pred_hbm, mask_hbm, out_hbm, mb0, mb1, pb0, pb1, obuf,
             sm0, sm1, sp0, sp1):
  cid = lax.axis_index("c")
  sid = lax.axis_index("s")
  w = sid * NC + cid
  c0 = w * CPW

  mbufs = (mb0, mb1)
  pbufs = (pb0, pb1)
  msems = (sm0, sm1)
  psems = (sp0, sp1)

  neg_inf = jnp.float32(-jnp.inf)
  zero = jnp.zeros((LANES,), jnp.float32)
  one = jnp.float32(1.0)

  def start(step):
    b, rem = divmod(step, CPW)
    par = step % 2
    c = c0 + rem
    xi = c // QS
    q = c % QS
    prow = b * NX + xi
    poff = prow * PLANE + q * VC
    moff = prow * CH * PLANE + q * VC
    copies = [
        pltpu.async_copy(mask_hbm.at[pl.ds(moff + ch * PLANE, VC)],
                         mbufs[par].at[ch], msems[par])
        for ch in range(CH)
    ]
    copies.append(pltpu.async_copy(pred_hbm.at[pl.ds(poff, VC)],
                                   pbufs[par], psems[par]))
    return copies

  pending = start(0)
  cnts = sums = maxs = None
  for step in range(TOT):
    b, k = divmod(step, CPW)
    par = step % 2
    if k == 0:
      cnts = [zero] * CH
      sums = {ch: zero for ch in _MEAN_CH}
      maxs = {ch: jnp.full((LANES,), neg_inf) for ch in _MAX_CH}
    nxt = start(step + 1) if step + 1 < TOT else None
    for c in pending:
      c.wait()
    pending = nxt
    mb = mbufs[par]
    pb = pbufs[par]

    def inner(g, carry, mb=mb, pb=pb):
      cnts = list(carry[:CH])
      sums = dict(zip(_MEAN_CH, carry[CH:CH + 4]))
      maxs = dict(zip(_MAX_CH, carry[CH + 4:]))
      off = g * LANES
      pv = pb[pl.ds(off, LANES)]
      for ch in range(CH):
        mv = mb[ch, pl.ds(off, LANES)]
        m = mv > 0.0
        cnts[ch] = cnts[ch] + jnp.where(m, one, 0.0)
        if ch in _MEAN_CH:
          sums[ch] = sums[ch] + jnp.where(m, pv, 0.0)
        else:
          maxs[ch] = jnp.maximum(maxs[ch], jnp.where(m, pv, neg_inf))
      return tuple(cnts) + tuple(sums[c] for c in _MEAN_CH) + tuple(
          maxs[c] for c in _MAX_CH)

    carry = tuple(cnts) + tuple(sums[c] for c in _MEAN_CH) + tuple(
        maxs[c] for c in _MAX_CH)
    carry = lax.fori_loop(0, GROUPS, inner, carry)
    cnts = list(carry[:CH])
    sums = dict(zip(_MEAN_CH, carry[CH:CH + 4]))
    maxs = dict(zip(_MAX_CH, carry[CH + 4:]))

    if k == CPW - 1:
      for ch in range(CH):
        r0 = (0 * B * CH + b * CH + ch) * LANES
        r1 = (1 * B * CH + b * CH + ch) * LANES
        r2 = (2 * B * CH + b * CH + ch) * LANES
        obuf[pl.ds(r0, LANES)] = sums[ch] if ch in _MEAN_CH else zero
        obuf[pl.ds(r1, LANES)] = cnts[ch]
        obuf[pl.ds(r2, LANES)] = (
            maxs[ch] if ch in _MAX_CH else jnp.full((LANES,), neg_inf))

  pltpu.sync_copy(obuf, out_hbm.at[pl.ds(w * P_FLAT, P_FLAT)])


_sc_partials = functools.partial(
    pl.kernel,
    out_type=jax.ShapeDtypeStruct((NW * P_FLAT,), jnp.float32),
    mesh=plsc.VectorSubcoreMesh(core_axis_name="c", subcore_axis_name="s",
                                num_cores=NC, num_subcores=NS),
    scratch_types=[
        pltpu.VMEM((CH, VC), jnp.float32),
        pltpu.VMEM((CH, VC), jnp.float32),
        pltpu.VMEM((VC,), jnp.float32),
        pltpu.VMEM((VC,), jnp.float32),
        pltpu.VMEM((P_FLAT,), jnp.float32),
        pltpu.SemaphoreType.DMA,
        pltpu.SemaphoreType.DMA,
        pltpu.SemaphoreType.DMA,
        pltpu.SemaphoreType.DMA,
    ],
    compiler_params=pltpu.CompilerParams(use_tc_tiling_on_sc=False,
                                         needs_layout_passes=False),
)(_sc_body)

_THRESH = {0: 54.0, 1: 48.0, 2: 26.0, 3: 26.0, 4: 45.0, 5: 45.0, 6: 73.5}


def _tc_body(pred_ref, mask_ref, out_ref):
  neg_inf = jnp.float32(-jnp.inf)
  vals = {}
  for xs in range(TC_XS):
    pv = pred_ref[xs]
    for ch in range(CH):
      mv = mask_ref[xs * CH + ch]
      m = mv > 0.0
      cnt = jnp.sum(jnp.where(m, 1.0, 0.0))
      vals[ch] = vals.get(ch, 0.0) + cnt
      if ch in _MEAN_CH:
        s = jnp.sum(jnp.where(m, pv, 0.0))
        vals[CH + ch] = vals.get(CH + ch, 0.0) + s
      else:
        mx = jnp.max(jnp.where(m, pv, neg_inf))
        vals[2 * CH + ch] = jnp.maximum(vals.get(2 * CH + ch, neg_inf), mx)
  col = lax.broadcasted_iota(jnp.int32, (1, 1, 3 * CH), 2)
  row = jnp.zeros((1, 1, 3 * CH), jnp.float32)
  for k, v in vals.items():
    row = jnp.where(col == k, v, row)
  out_ref[...] = row


_TC_STEPS = X_TC // TC_XS

_tc_call = pl.pallas_call(
    _tc_body,
    grid=(B, _TC_STEPS),
    in_specs=[
        pl.BlockSpec((TC_XS, 128, 128),
                     lambda b, x: ((b * NX + X_SC) // TC_XS + x, 0, 0)),
        pl.BlockSpec((TC_XS * CH, 128, 128),
                     lambda b, x: ((b * NX + X_SC) // TC_XS + x, 0, 0)),
    ],
    out_specs=pl.BlockSpec((1, 1, 3 * CH),
                           lambda b, x: (b * _TC_STEPS + x, 0, 0)),
    out_shape=jax.ShapeDtypeStruct((B * _TC_STEPS, 1, 3 * CH), jnp.float32),
)


def _combine(p_ref, t_ref, o_ref):
  p = p_ref[...]
  t = t_ref[...][:, 0, :]

  def seg_sum(stat, b, ch):
    off = (stat * B * CH + b * CH + ch) * LANES
    return jnp.sum(p[:, off:off + LANES])

  def seg_max(stat, b, ch):
    off = (stat * B * CH + b * CH + ch) * LANES
    return jnp.max(p[:, off:off + LANES])

  def tc_sum(col, b):
    return jnp.sum(t[b * _TC_STEPS:(b + 1) * _TC_STEPS, col:col + 1])

  def tc_max(col, b):
    return jnp.max(t[b * _TC_STEPS:(b + 1) * _TC_STEPS, col:col + 1])

  total = jnp.float32(0.0)
  count = jnp.float32(0.0)
  for ch in range(CH):
    struct_loss = jnp.float32(0.0)
    present_any = jnp.float32(0.0)
    for b in range(B):
      cnt = seg_sum(1, b, ch) + tc_sum(ch, b)
      present = cnt > 0.0
      if ch in _MAX_CH:
        stat = jnp.maximum(seg_max(2, b, ch), tc_max(2 * CH + ch, b))
      else:
        stat = (seg_sum(0, b, ch) + tc_sum(CH + ch, b)) / jnp.maximum(cnt, 1.0)
      loss_b = jnp.where(stat <= jnp.float32(_THRESH[ch]), 0.0, 1.0)
      struct_loss = struct_loss + jnp.where(present, loss_b, 0.0)
      present_any = jnp.maximum(present_any,
                                jnp.where(present, 1.0, 0.0))
    total = total + struct_loss
    count = count + present_any
  o_ref[0] = total / jnp.maximum(count, 1.0)


_combine_call = pl.pallas_call(
    _combine,
    out_shape=jax.ShapeDtypeStruct((1,), jnp.float32),
    out_specs=pl.BlockSpec(memory_space=pltpu.SMEM),
)


def kernel(predicted, structure_masks):
  pred_flat = predicted.reshape(-1)
  mask_nat = structure_masks.transpose(0, 1, 4, 2, 3)
  mask_lin = mask_nat.reshape(-1)
  pred3d = predicted.reshape(B * NX, 128, 128)
  mask3d = mask_nat.reshape(B * NX * CH, 128, 128)
  partials = _sc_partials(pred_flat, mask_lin)
  tparts = _tc_call(pred3d, mask3d)
  partials = partials.reshape(NW, P_FLAT)
  return _combine_call(partials, tparts).astype(predicted.dtype)


# speedup vs baseline: 1.6173x; 1.0330x over previous
import functools

import jax
import jax.numpy as jnp
from jax import lax
from jax.experimental import pallas as pl
from jax.experimental.pallas import tpu as pltpu
from jax.experimental.pallas import tpu_sc as plsc

NC = 2
NS = 16
NW = NC * NS
LANES = 16

B = 2
NX = 128
PLANE = 128 * 128
NVOX = NX * PLANE
CH = 7
X_SC = 64
X_TC = NX - X_SC
TC_XS = 8
QS = 4
VC = PLANE // QS
CPW = X_SC * QS // NW
TOT = B * CPW
GROUPS = VC // LANES

_MAX_CH = (0, 1, 6)
_MEAN_CH = (2, 3, 4, 5)
P_ROWS = 3 * B * CH
P_FLAT = P_ROWS * LANES


def _sc_body(pred_hbm, mask_hbm, out_hbm, mb0, mb1, pb0, pb1, obuf,
             sm0, sm1, sp0, sp1):
  cid = lax.axis_index("c")
  sid = lax.axis_index("s")
  w = sid * NC + cid
  c0 = w * CPW

  mbufs = (mb0, mb1)
  pbufs = (pb0, pb1)
  msems = (sm0, sm1)
  psems = (sp0, sp1)

  neg_inf = jnp.float32(-jnp.inf)
  zero = jnp.zeros((LANES,), jnp.float32)
  one = jnp.float32(1.0)

  def start(step):
    b, rem = divmod(step, CPW)
    par = step % 2
    c = c0 + rem
    xi = c // QS
    q = c % QS
    prow = b * NX + xi
    poff = prow * PLANE + q * VC
    moff = prow * CH * PLANE + q * VC
    copies = [
        pltpu.async_copy(mask_hbm.at[pl.ds(moff + ch * PLANE, VC)],
                         mbufs[par].at[ch], msems[par])
        for ch in range(CH)
    ]
    copies.append(pltpu.async_copy(pred_hbm.at[pl.ds(poff, VC)],
                                   pbufs[par], psems[par]))
    return copies

  pending = start(0)
  cnts = sums = maxs = None
  for step in range(TOT):
    b, k = divmod(step, CPW)
    par = step % 2
    if k == 0:
      cnts = [zero] * CH
      sums = {ch: zero for ch in _MEAN_CH}
      maxs = {ch: jnp.full((LANES,), neg_inf) for ch in _MAX_CH}
    nxt = start(step + 1) if step + 1 < TOT else None
    for c in pending:
      c.wait()
    pending = nxt
    mb = mbufs[par]
    pb = pbufs[par]

    def inner(g, carry, mb=mb, pb=pb):
      cnts = list(carry[:CH])
      sums = dict(zip(_MEAN_CH, carry[CH:CH + 4]))
      maxs = dict(zip(_MAX_CH, carry[CH + 4:]))
      off = g * LANES
      pv = pb[pl.ds(off, LANES)]
      for ch in range(CH):
        mv = mb[ch, pl.ds(off, LANES)]
        m = mv > 0.0
        cnts[ch] = cnts[ch] + jnp.where(m, one, 0.0)
        if ch in _MEAN_CH:
          sums[ch] = sums[ch] + jnp.where(m, pv, 0.0)
        else:
          maxs[ch] = jnp.maximum(maxs[ch], jnp.where(m, pv, neg_inf))
      return tuple(cnts) + tuple(sums[c] for c in _MEAN_CH) + tuple(
          maxs[c] for c in _MAX_CH)

    carry = tuple(cnts) + tuple(sums[c] for c in _MEAN_CH) + tuple(
        maxs[c] for c in _MAX_CH)
    carry = lax.fori_loop(0, GROUPS, inner, carry)
    cnts = list(carry[:CH])
    sums = dict(zip(_MEAN_CH, carry[CH:CH + 4]))
    maxs = dict(zip(_MAX_CH, carry[CH + 4:]))

    if k == CPW - 1:
      for ch in range(CH):
        r0 = (0 * B * CH + b * CH + ch) * LANES
        r1 = (1 * B * CH + b * CH + ch) * LANES
        r2 = (2 * B * CH + b * CH + ch) * LANES
        obuf[pl.ds(r0, LANES)] = sums[ch] if ch in _MEAN_CH else zero
        obuf[pl.ds(r1, LANES)] = cnts[ch]
        obuf[pl.ds(r2, LANES)] = (
            maxs[ch] if ch in _MAX_CH else jnp.full((LANES,), neg_inf))

  pltpu.sync_copy(obuf, out_hbm.at[pl.ds(w * P_FLAT, P_FLAT)])


_sc_partials = functools.partial(
    pl.kernel,
    out_type=jax.ShapeDtypeStruct((NW * P_FLAT,), jnp.float32),
    mesh=plsc.VectorSubcoreMesh(core_axis_name="c", subcore_axis_name="s",
                                num_cores=NC, num_subcores=NS),
    scratch_types=[
        pltpu.VMEM((CH, VC), jnp.float32),
        pltpu.VMEM((CH, VC), jnp.float32),
        pltpu.VMEM((VC,), jnp.float32),
        pltpu.VMEM((VC,), jnp.float32),
        pltpu.VMEM((P_FLAT,), jnp.float32),
        pltpu.SemaphoreType.DMA,
        pltpu.SemaphoreType.DMA,
        pltpu.SemaphoreType.DMA,
        pltpu.SemaphoreType.DMA,
    ],
    compiler_params=pltpu.CompilerParams(use_tc_tiling_on_sc=False,
                                         needs_layout_passes=False),
)(_sc_body)

_THRESH = {0: 54.0, 1: 48.0, 2: 26.0, 3: 26.0, 4: 45.0, 5: 45.0, 6: 73.5}


def _tc_body(pred_ref, mask_ref, out_ref):
  neg_inf = jnp.float32(-jnp.inf)
  vals = {}
  for xs in range(TC_XS):
    pv = pred_ref[xs]
    for ch in range(CH):
      mv = mask_ref[xs * CH + ch]
      m = mv > 0.0
      cnt = jnp.sum(jnp.where(m, 1.0, 0.0))
      vals[ch] = vals.get(ch, 0.0) + cnt
      if ch in _MEAN_CH:
        s = jnp.sum(jnp.where(m, pv, 0.0))
        vals[CH + ch] = vals.get(CH + ch, 0.0) + s
      else:
        mx = jnp.max(jnp.where(m, pv, neg_inf))
        vals[2 * CH + ch] = jnp.maximum(vals.get(2 * CH + ch, neg_inf), mx)
  col = lax.broadcasted_iota(jnp.int32, (1, 1, 3 * CH), 2)
  row = jnp.zeros((1, 1, 3 * CH), jnp.float32)
  for k, v in vals.items():
    row = jnp.where(col == k, v, row)
  out_ref[...] = row


_TC_STEPS = X_TC // TC_XS

_tc_call = pl.pallas_call(
    _tc_body,
    grid=(B, _TC_STEPS),
    in_specs=[
        pl.BlockSpec((TC_XS, 128, 128),
                     lambda b, x: ((b * NX + X_SC) // TC_XS + x, 0, 0)),
        pl.BlockSpec((TC_XS * CH, 128, 128),
                     lambda b, x: ((b * NX + X_SC) // TC_XS + x, 0, 0)),
    ],
    out_specs=pl.BlockSpec((1, 1, 3 * CH),
                           lambda b, x: (b * _TC_STEPS + x, 0, 0)),
    out_shape=jax.ShapeDtypeStruct((B * _TC_STEPS, 1, 3 * CH), jnp.float32),
)


def _combine(p_ref, t_ref, o_ref):
  p = p_ref[...]
  t = t_ref[...][:, 0, :]

  def seg_sum(stat, b, ch):
    off = (stat * B * CH + b * CH + ch) * LANES
    return jnp.sum(p[:, off:off + LANES])

  def seg_max(stat, b, ch):
    off = (stat * B * CH + b * CH + ch) * LANES
    return jnp.max(p[:, off:off + LANES])

  def tc_sum(col, b):
    return jnp.sum(t[b * _TC_STEPS:(b + 1) * _TC_STEPS, col:col + 1])

  def tc_max(col, b):
    return jnp.max(t[b * _TC_STEPS:(b + 1) * _TC_STEPS, col:col + 1])

  total = jnp.float32(0.0)
  count = jnp.float32(0.0)
  for ch in range(CH):
    struct_loss = jnp.float32(0.0)
    present_any = jnp.float32(0.0)
    for b in range(B):
      cnt = seg_sum(1, b, ch) + tc_sum(ch, b)
      present = cnt > 0.0
      if ch in _MAX_CH:
        stat = jnp.maximum(seg_max(2, b, ch), tc_max(2 * CH + ch, b))
      else:
        stat = (seg_sum(0, b, ch) + tc_sum(CH + ch, b)) / jnp.maximum(cnt, 1.0)
      loss_b = jnp.where(stat <= jnp.float32(_THRESH[ch]), 0.0, 1.0)
      struct_loss = struct_loss + jnp.where(present, loss_b, 0.0)
      present_any = jnp.maximum(present_any,
                                jnp.where(present, 1.0, 0.0))
    total = total + struct_loss
    count = count + present_any
  o_ref[0] = total / jnp.maximum(count, 1.0)


_combine_call = pl.pallas_call(
    _combine,
    out_shape=jax.ShapeDtypeStruct((1,), jnp.float32),
    out_specs=pl.BlockSpec(memory_space=pltpu.SMEM),
)


def kernel(predicted, structure_masks):
  pred_flat = predicted.reshape(-1)
  mask_nat = structure_masks.transpose(0, 1, 4, 2, 3)
  mask_lin = mask_nat.reshape(-1)
  pred3d = predicted.reshape(B * NX, 128, 128)
  mask3d = mask_nat.reshape(B * NX * CH, 128, 128)
  partials = _sc_partials(pred_flat, mask_lin)
  tparts = _tc_call(pred3d, mask3d)
  partials = partials.reshape(NW, P_FLAT)
  return _combine_call(partials, tparts).astype(predicted.dtype)


# speedup vs baseline: 1.6285x; 1.0069x over previous
import functools

import jax
import jax.numpy as jnp
from jax import lax
from jax.experimental import pallas as pl
from jax.experimental.pallas import tpu as pltpu
from jax.experimental.pallas import tpu_sc as plsc

NC = 2
NS = 16
NW = NC * NS
LANES = 16

B = 2
NX = 128
PLANE = 128 * 128
NVOX = NX * PLANE
CH = 7
X_SC = 64
X_TC = NX - X_SC
TC_XS = 16
QS = 4
VC = PLANE // QS
CPW = X_SC * QS // NW
TOT = B * CPW
GROUPS = VC // LANES

_MAX_CH = (0, 1, 6)
_MEAN_CH = (2, 3, 4, 5)
P_ROWS = 3 * B * CH
P_FLAT = P_ROWS * LANES


def _sc_body(pred_hbm, mask_hbm, out_hbm, mb0, mb1, pb0, pb1, obuf,
             sm0, sm1, sp0, sp1):
  cid = lax.axis_index("c")
  sid = lax.axis_index("s")
  w = sid * NC + cid
  c0 = w * CPW

  mbufs = (mb0, mb1)
  pbufs = (pb0, pb1)
  msems = (sm0, sm1)
  psems = (sp0, sp1)

  neg_inf = jnp.float32(-jnp.inf)
  zero = jnp.zeros((LANES,), jnp.float32)
  one = jnp.float32(1.0)

  def start(step):
    b, rem = divmod(step, CPW)
    par = step % 2
    c = c0 + rem
    xi = c // QS
    q = c % QS
    prow = b * NX + xi
    poff = prow * PLANE + q * VC
    moff = prow * CH * PLANE + q * VC
    copies = [
        pltpu.async_copy(mask_hbm.at[pl.ds(moff + ch * PLANE, VC)],
                         mbufs[par].at[ch], msems[par])
        for ch in range(CH)
    ]
    copies.append(pltpu.async_copy(pred_hbm.at[pl.ds(poff, VC)],
                                   pbufs[par], psems[par]))
    return copies

  pending = start(0)
  cnts = sums = maxs = None
  for step in range(TOT):
    b, k = divmod(step, CPW)
    par = step % 2
    if k == 0:
      cnts = [zero] * CH
      sums = {ch: zero for ch in _MEAN_CH}
      maxs = {ch: jnp.full((LANES,), neg_inf) for ch in _MAX_CH}
    nxt = start(step + 1) if step + 1 < TOT else None
    for c in pending:
      c.wait()
    pending = nxt
    mb = mbufs[par]
    pb = pbufs[par]

    def inner(g, carry, mb=mb, pb=pb):
      cnts = list(carry[:CH])
      sums = dict(zip(_MEAN_CH, carry[CH:CH + 4]))
      maxs = dict(zip(_MAX_CH, carry[CH + 4:]))
      off = g * LANES
      pv = pb[pl.ds(off, LANES)]
      for ch in range(CH):
        mv = mb[ch, pl.ds(off, LANES)]
        m = mv > 0.0
        cnts[ch] = cnts[ch] + jnp.where(m, one, 0.0)
        if ch in _MEAN_CH:
          sums[ch] = sums[ch] + jnp.where(m, pv, 0.0)
        else:
          maxs[ch] = jnp.maximum(maxs[ch], jnp.where(m, pv, neg_inf))
      return tuple(cnts) + tuple(sums[c] for c in _MEAN_CH) + tuple(
          maxs[c] for c in _MAX_CH)

    carry = tuple(cnts) + tuple(sums[c] for c in _MEAN_CH) + tuple(
        maxs[c] for c in _MAX_CH)
    carry = lax.fori_loop(0, GROUPS, inner, carry)
    cnts = list(carry[:CH])
    sums = dict(zip(_MEAN_CH, carry[CH:CH + 4]))
    maxs = dict(zip(_MAX_CH, carry[CH + 4:]))

    if k == CPW - 1:
      for ch in range(CH):
        r0 = (0 * B * CH + b * CH + ch) * LANES
        r1 = (1 * B * CH + b * CH + ch) * LANES
        r2 = (2 * B * CH + b * CH + ch) * LANES
        obuf[pl.ds(r0, LANES)] = sums[ch] if ch in _MEAN_CH else zero
        obuf[pl.ds(r1, LANES)] = cnts[ch]
        obuf[pl.ds(r2, LANES)] = (
            maxs[ch] if ch in _MAX_CH else jnp.full((LANES,), neg_inf))

  pltpu.sync_copy(obuf, out_hbm.at[pl.ds(w * P_FLAT, P_FLAT)])


_sc_partials = functools.partial(
    pl.kernel,
    out_type=jax.ShapeDtypeStruct((NW * P_FLAT,), jnp.float32),
    mesh=plsc.VectorSubcoreMesh(core_axis_name="c", subcore_axis_name="s",
                                num_cores=NC, num_subcores=NS),
    scratch_types=[
        pltpu.VMEM((CH, VC), jnp.float32),
        pltpu.VMEM((CH, VC), jnp.float32),
        pltpu.VMEM((VC,), jnp.float32),
        pltpu.VMEM((VC,), jnp.float32),
        pltpu.VMEM((P_FLAT,), jnp.float32),
        pltpu.SemaphoreType.DMA,
        pltpu.SemaphoreType.DMA,
        pltpu.SemaphoreType.DMA,
        pltpu.SemaphoreType.DMA,
    ],
    compiler_params=pltpu.CompilerParams(use_tc_tiling_on_sc=False,
                                         needs_layout_passes=False),
)(_sc_body)

_THRESH = {0: 54.0, 1: 48.0, 2: 26.0, 3: 26.0, 4: 45.0, 5: 45.0, 6: 73.5}


def _tc_body(pred_ref, mask_ref, out_ref):
  neg_inf = jnp.float32(-jnp.inf)
  vals = {}
  for xs in range(TC_XS):
    pv = pred_ref[xs]
    for ch in range(CH):
      mv = mask_ref[xs * CH + ch]
      m = mv > 0.0
      cnt = jnp.sum(jnp.where(m, 1.0, 0.0))
      vals[ch] = vals.get(ch, 0.0) + cnt
      if ch in _MEAN_CH:
        s = jnp.sum(jnp.where(m, pv, 0.0))
        vals[CH + ch] = vals.get(CH + ch, 0.0) + s
      else:
        mx = jnp.max(jnp.where(m, pv, neg_inf))
        vals[2 * CH + ch] = jnp.maximum(vals.get(2 * CH + ch, neg_inf), mx)
  col = lax.broadcasted_iota(jnp.int32, (1, 1, 3 * CH), 2)
  row = jnp.zeros((1, 1, 3 * CH), jnp.float32)
  for k, v in vals.items():
    row = jnp.where(col == k, v, row)
  out_ref[...] = row


_TC_STEPS = X_TC // TC_XS

_tc_call = pl.pallas_call(
    _tc_body,
    grid=(B, _TC_STEPS),
    in_specs=[
        pl.BlockSpec((TC_XS, 128, 128),
                     lambda b, x: ((b * NX + X_SC) // TC_XS + x, 0, 0)),
        pl.BlockSpec((TC_XS * CH, 128, 128),
                     lambda b, x: ((b * NX + X_SC) // TC_XS + x, 0, 0)),
    ],
    out_specs=pl.BlockSpec((1, 1, 3 * CH),
                           lambda b, x: (b * _TC_STEPS + x, 0, 0)),
    out_shape=jax.ShapeDtypeStruct((B * _TC_STEPS, 1, 3 * CH), jnp.float32),
)


def _combine(p_ref, t_ref, o_ref):
  p = p_ref[...]
  t = t_ref[...][:, 0, :]

  def seg_sum(stat, b, ch):
    off = (stat * B * CH + b * CH + ch) * LANES
    return jnp.sum(p[:, off:off + LANES])

  def seg_max(stat, b, ch):
    off = (stat * B * CH + b * CH + ch) * LANES
    return jnp.max(p[:, off:off + LANES])

  def tc_sum(col, b):
    return jnp.sum(t[b * _TC_STEPS:(b + 1) * _TC_STEPS, col:col + 1])

  def tc_max(col, b):
    return jnp.max(t[b * _TC_STEPS:(b + 1) * _TC_STEPS, col:col + 1])

  total = jnp.float32(0.0)
  count = jnp.float32(0.0)
  for ch in range(CH):
    struct_loss = jnp.float32(0.0)
    present_any = jnp.float32(0.0)
    for b in range(B):
      cnt = seg_sum(1, b, ch) + tc_sum(ch, b)
      present = cnt > 0.0
      if ch in _MAX_CH:
        stat = jnp.maximum(seg_max(2, b, ch), tc_max(2 * CH + ch, b))
      else:
        stat = (seg_sum(0, b, ch) + tc_sum(CH + ch, b)) / jnp.maximum(cnt, 1.0)
      loss_b = jnp.where(stat <= jnp.float32(_THRESH[ch]), 0.0, 1.0)
      struct_loss = struct_loss + jnp.where(present, loss_b, 0.0)
      present_any = jnp.maximum(present_any,
                                jnp.where(present, 1.0, 0.0))
    total = total + struct_loss
    count = count + present_any
  o_ref[0] = total / jnp.maximum(count, 1.0)


_combine_call = pl.pallas_call(
    _combine,
    out_shape=jax.ShapeDtypeStruct((1,), jnp.float32),
    out_specs=pl.BlockSpec(memory_space=pltpu.SMEM),
)


def kernel(predicted, structure_masks):
  pred_flat = predicted.reshape(-1)
  mask_nat = structure_masks.transpose(0, 1, 4, 2, 3)
  mask_lin = mask_nat.reshape(-1)
  pred3d = predicted.reshape(B * NX, 128, 128)
  mask3d = mask_nat.reshape(B * NX * CH, 128, 128)
  partials = _sc_partials(pred_flat, mask_lin)
  tparts = _tc_call(pred3d, mask3d)
  partials = partials.reshape(NW, P_FLAT)
  return _combine_call(partials, tparts).astype(predicted.dtype)


# speedup vs baseline: 1.6576x; 1.0178x over previous
import functools

import jax
import jax.numpy as jnp
from jax import lax
from jax.experimental import pallas as pl
from jax.experimental.pallas import tpu as pltpu
from jax.experimental.pallas import tpu_sc as plsc

NC = 2
NS = 16
NW = NC * NS
LANES = 16

B = 2
NX = 128
PLANE = 128 * 128
NVOX = NX * PLANE
CH = 7
X_SC = 64
X_TC = NX - X_SC
TC_XS = 16
QS = 4
VC = PLANE // QS
CPW = X_SC * QS // NW
TOT = B * CPW
GROUPS = VC // LANES

_MAX_CH = (0, 1, 6)
_MEAN_CH = (2, 3, 4, 5)
P_FLAT = 4 * LANES


def _sc_body(pred_hbm, mask_hbm, out_hbm, mb0, mb1, pb0, pb1, obuf,
             sm0, sm1, sp0, sp1):
  cid = lax.axis_index("c")
  sid = lax.axis_index("s")
  w = sid * NC + cid
  c0 = w * CPW

  mbufs = (mb0, mb1)
  pbufs = (pb0, pb1)
  msems = (sm0, sm1)
  psems = (sp0, sp1)

  neg_inf = jnp.float32(-jnp.inf)
  zero = jnp.zeros((LANES,), jnp.float32)
  one = jnp.float32(1.0)

  def start(step):
    b, rem = divmod(step, CPW)
    par = step % 2
    c = c0 + rem
    xi = c // QS
    q = c % QS
    prow = b * NX + xi
    poff = prow * PLANE + q * VC
    moff = prow * CH * PLANE + q * VC
    copies = [
        pltpu.async_copy(mask_hbm.at[pl.ds(moff + ch * PLANE, VC)],
                         mbufs[par].at[ch], msems[par])
        for ch in range(CH)
    ]
    copies.append(pltpu.async_copy(pred_hbm.at[pl.ds(poff, VC)],
                                   pbufs[par], psems[par]))
    return copies

  pending = start(0)
  cnts = sums = maxs = None
  psum = [zero, zero, jnp.full((LANES,), neg_inf)]
  for step in range(TOT):
    b, k = divmod(step, CPW)
    par = step % 2
    if k == 0:
      cnts = [zero] * CH
      sums = {ch: zero for ch in _MEAN_CH}
      maxs = {ch: jnp.full((LANES,), neg_inf) for ch in _MAX_CH}
    nxt = start(step + 1) if step + 1 < TOT else None
    for c in pending:
      c.wait()
    pending = nxt
    mb = mbufs[par]
    pb = pbufs[par]

    def inner(g, carry, mb=mb, pb=pb):
      cnts = list(carry[:CH])
      sums = dict(zip(_MEAN_CH, carry[CH:CH + 4]))
      maxs = dict(zip(_MAX_CH, carry[CH + 4:]))
      off = g * LANES
      pv = pb[pl.ds(off, LANES)]
      for ch in range(CH):
        mv = mb[ch, pl.ds(off, LANES)]
        m = mv > 0.0
        cnts[ch] = cnts[ch] + jnp.where(m, one, 0.0)
        if ch in _MEAN_CH:
          sums[ch] = sums[ch] + jnp.where(m, pv, 0.0)
        else:
          maxs[ch] = jnp.maximum(maxs[ch], jnp.where(m, pv, neg_inf))
      return tuple(cnts) + tuple(sums[c] for c in _MEAN_CH) + tuple(
          maxs[c] for c in _MAX_CH)

    carry = tuple(cnts) + tuple(sums[c] for c in _MEAN_CH) + tuple(
        maxs[c] for c in _MAX_CH)
    carry = lax.fori_loop(0, GROUPS, inner, carry)
    cnts = list(carry[:CH])
    sums = dict(zip(_MEAN_CH, carry[CH:CH + 4]))
    maxs = dict(zip(_MAX_CH, carry[CH + 4:]))

    if k == CPW - 1:
      lane = lax.iota(jnp.int32, LANES)
      for ch in range(CH):
        idx = b * CH + ch
        psum[0] = jnp.where(
            lane == idx,
            jnp.sum(sums[ch]) if ch in _MEAN_CH else 0.0, psum[0])
        psum[1] = jnp.where(lane == idx, jnp.sum(cnts[ch]), psum[1])
        psum[2] = jnp.where(
            lane == idx,
            jnp.max(maxs[ch]) if ch in _MAX_CH else neg_inf, psum[2])

  for stat in range(3):
    obuf[pl.ds(stat * LANES, LANES)] = psum[stat]
  obuf[pl.ds(3 * LANES, LANES)] = zero
  pltpu.sync_copy(obuf, out_hbm.at[pl.ds(w * P_FLAT, P_FLAT)])


_sc_partials = functools.partial(
    pl.kernel,
    out_type=jax.ShapeDtypeStruct((NW * P_FLAT,), jnp.float32),
    mesh=plsc.VectorSubcoreMesh(core_axis_name="c", subcore_axis_name="s",
                                num_cores=NC, num_subcores=NS),
    scratch_types=[
        pltpu.VMEM((CH, VC), jnp.float32),
        pltpu.VMEM((CH, VC), jnp.float32),
        pltpu.VMEM((VC,), jnp.float32),
        pltpu.VMEM((VC,), jnp.float32),
        pltpu.VMEM((P_FLAT,), jnp.float32),
        pltpu.SemaphoreType.DMA,
        pltpu.SemaphoreType.DMA,
        pltpu.SemaphoreType.DMA,
        pltpu.SemaphoreType.DMA,
    ],
    compiler_params=pltpu.CompilerParams(use_tc_tiling_on_sc=False,
                                         needs_layout_passes=False),
)(_sc_body)

_THRESH = {0: 54.0, 1: 48.0, 2: 26.0, 3: 26.0, 4: 45.0, 5: 45.0, 6: 73.5}


def _tc_body(pred_ref, mask_ref, out_ref):
  neg_inf = jnp.float32(-jnp.inf)
  vals = {}
  for xs in range(TC_XS):
    pv = pred_ref[xs]
    for ch in range(CH):
      mv = mask_ref[xs * CH + ch]
      m = mv > 0.0
      cnt = jnp.sum(jnp.where(m, 1.0, 0.0))
      vals[ch] = vals.get(ch, 0.0) + cnt
      if ch in _MEAN_CH:
        s = jnp.sum(jnp.where(m, pv, 0.0))
        vals[CH + ch] = vals.get(CH + ch, 0.0) + s
      else:
        mx = jnp.max(jnp.where(m, pv, neg_inf))
        vals[2 * CH + ch] = jnp.maximum(vals.get(2 * CH + ch, neg_inf), mx)
  col = lax.broadcasted_iota(jnp.int32, (1, 1, 3 * CH), 2)
  row = jnp.zeros((1, 1, 3 * CH), jnp.float32)
  for k, v in vals.items():
    row = jnp.where(col == k, v, row)
  out_ref[...] = row


_TC_STEPS = X_TC // TC_XS

_tc_call = pl.pallas_call(
    _tc_body,
    grid=(B, _TC_STEPS),
    in_specs=[
        pl.BlockSpec((TC_XS, 128, 128),
                     lambda b, x: ((b * NX + X_SC) // TC_XS + x, 0, 0)),
        pl.BlockSpec((TC_XS * CH, 128, 128),
                     lambda b, x: ((b * NX + X_SC) // TC_XS + x, 0, 0)),
    ],
    out_specs=pl.BlockSpec((1, 1, 3 * CH),
                           lambda b, x: (b * _TC_STEPS + x, 0, 0)),
    out_shape=jax.ShapeDtypeStruct((B * _TC_STEPS, 1, 3 * CH), jnp.float32),
)


def _combine(p_ref, t_ref, o_ref):
  p = p_ref[...]
  t = t_ref[...][:, 0, :]

  def seg_sum(stat, b, ch):
    c = stat * LANES + b * CH + ch
    return jnp.sum(p[:, c:c + 1]) + jnp.sum(p[:, P_FLAT + c:P_FLAT + c + 1])

  def seg_max(stat, b, ch):
    c = stat * LANES + b * CH + ch
    return jnp.maximum(jnp.max(p[:, c:c + 1]),
                       jnp.max(p[:, P_FLAT + c:P_FLAT + c + 1]))

  def tc_sum(col, b):
    return jnp.sum(t[b * _TC_STEPS:(b + 1) * _TC_STEPS, col:col + 1])

  def tc_max(col, b):
    return jnp.max(t[b * _TC_STEPS:(b + 1) * _TC_STEPS, col:col + 1])

  total = jnp.float32(0.0)
  count = jnp.float32(0.0)
  for ch in range(CH):
    struct_loss = jnp.float32(0.0)
    present_any = jnp.float32(0.0)
    for b in range(B):
      cnt = seg_sum(1, b, ch) + tc_sum(ch, b)
      present = cnt > 0.0
      if ch in _MAX_CH:
        stat = jnp.maximum(seg_max(2, b, ch), tc_max(2 * CH + ch, b))
      else:
        stat = (seg_sum(0, b, ch) + tc_sum(CH + ch, b)) / jnp.maximum(cnt, 1.0)
      loss_b = jnp.where(stat <= jnp.float32(_THRESH[ch]), 0.0, 1.0)
      struct_loss = struct_loss + jnp.where(present, loss_b, 0.0)
      present_any = jnp.maximum(present_any,
                                jnp.where(present, 1.0, 0.0))
    total = total + struct_loss
    count = count + present_any
  o_ref[0] = total / jnp.maximum(count, 1.0)


_combine_call = pl.pallas_call(
    _combine,
    out_shape=jax.ShapeDtypeStruct((1,), jnp.float32),
    out_specs=pl.BlockSpec(memory_space=pltpu.SMEM),
)


def kernel(predicted, structure_masks):
  pred_flat = predicted.reshape(-1)
  mask_nat = structure_masks.transpose(0, 1, 4, 2, 3)
  mask_lin = mask_nat.reshape(-1)
  pred3d = predicted.reshape(B * NX, 128, 128)
  mask3d = mask_nat.reshape(B * NX * CH, 128, 128)
  partials = _sc_partials(pred_flat, mask_lin)
  tparts = _tc_call(pred3d, mask3d)
  partials = partials.reshape(NW // 2, 2 * P_FLAT)
  return _combine_call(partials, tparts).astype(predicted.dtype)
